# Initial kernel scaffold; baseline (speedup 1.0000x reference)
import jax
import jax.numpy as jnp
from jax.experimental import pallas as pl


def kernel(x, Wr, W1, b1, W2, b2, W3, b3):
    raise NotImplementedError("write your pallas kernel here")


# speedup vs baseline: 1.2831x; 1.2831x over previous
import functools

import jax
import jax.numpy as jnp
from jax import lax
from jax.experimental import pallas as pl
from jax.experimental.pallas import tpu as pltpu
from jax.experimental.pallas import tpu_sc as plsc

DM = 768
DH = 3072
NE = 8
NTOK = 2048
NA = 2 * NTOK
BLK = 128
NBLK = (NA + NE * (BLK - 1) + BLK - 1) // BLK
NPAD = NBLK * BLK
CH = 512
NW = 32
TPW = NTOK // NW


def _router_body(x_ref, wr_ref, pos_ref, wexp_ref, blk_ref, aux_ref):
    x = x_ref[...]
    wr = wr_ref[...]
    logits = lax.dot_general(x, wr, (((1,), (1,)), ((), ())),
                             preferred_element_type=jnp.float32)
    m = jnp.max(logits, axis=1, keepdims=True)
    ex = jnp.exp(logits - m)
    p = ex / jnp.sum(ex, axis=1, keepdims=True)

    idx = lax.broadcasted_iota(jnp.int32, (NTOK, NE), 1)
    v1 = jnp.max(p, axis=1, keepdims=True)
    i1 = jnp.min(jnp.where(p == v1, idx, NE), axis=1, keepdims=True)
    p2 = jnp.where(idx == i1, -1.0, p)
    v2 = jnp.max(p2, axis=1, keepdims=True)
    i2 = jnp.min(jnp.where(p2 == v2, idx, NE), axis=1, keepdims=True)
    s = v1 + v2 + 1e-9
    w1 = v1 / s
    w2 = v2 / s

    e1 = (idx == i1).astype(jnp.float32)
    e2 = (idx == i2).astype(jnp.float32)
    oh = jnp.concatenate([e1, e2], axis=0)

    ri = lax.broadcasted_iota(jnp.int32, (CH, CH), 0)
    ci = lax.broadcasted_iota(jnp.int32, (CH, CH), 1)
    tril = (ci < ri).astype(jnp.float32)
    tot = jnp.zeros((1, NE), jnp.float32)
    ranks = []
    for i in range(NA // CH):
        chunk = lax.slice(oh, (i * CH, 0), ((i + 1) * CH, NE))
        rk = lax.dot_general(tril, chunk, (((1,), (0,)), ((), ())),
                             preferred_element_type=jnp.float32) + tot
        ranks.append(rk)
        tot = tot + jnp.sum(chunk, axis=0, keepdims=True)
    rank = jnp.concatenate(ranks, axis=0)

    cnt = tot.astype(jnp.int32)
    pc = ((cnt + BLK - 1) // BLK) * BLK
    pcf = pc.astype(jnp.float32)
    er = lax.broadcasted_iota(jnp.int32, (NE, NE), 0)
    ec = lax.broadcasted_iota(jnp.int32, (NE, NE), 1)
    excl = (er < ec).astype(jnp.float32)
    offs = lax.dot_general(pcf, excl, (((1,), (0,)), ((), ())),
                           preferred_element_type=jnp.float32)

    posf = jnp.sum(oh * (rank + offs), axis=1, keepdims=True)
    pos_ref[...] = posf.astype(jnp.int32)

    w12 = jnp.concatenate([w1, w2], axis=0)
    wexp_ref[...] = jnp.broadcast_to(w12, (NA, 16))

    bno = lax.broadcasted_iota(jnp.int32, (NBLK, 1), 0)
    endb = (offs.astype(jnp.int32) + pc) // BLK
    emap = jnp.sum((bno >= endb).astype(jnp.int32), axis=1, keepdims=True)
    blk_ref[...] = jnp.minimum(emap, NE - 1)

    cnt1 = jnp.sum(e1, axis=0, keepdims=True)
    meanp = jnp.sum(p, axis=0, keepdims=True) * (1.0 / NTOK)
    aux_ref[...] = jnp.sum(cnt1 * meanp, axis=1, keepdims=True) * (
        float(NE) / NTOK)


def _router(x2d, wr):
    return pl.pallas_call(
        _router_body,
        out_shape=[
            jax.ShapeDtypeStruct((NA, 1), jnp.int32),
            jax.ShapeDtypeStruct((NA, 16), jnp.float32),
            jax.ShapeDtypeStruct((NBLK, 1), jnp.int32),
            jax.ShapeDtypeStruct((1, 1), jnp.float32),
        ],
    )(x2d, wr)


def _ffn_body(bmap_ref, xs_ref, w1_ref, w2_ref, w3_ref,
              b1_ref, b2_ref, b3_ref, ys_ref):
    del bmap_ref
    xb = xs_ref[...].astype(jnp.bfloat16)
    h1 = lax.dot_general(xb, w1_ref[0], (((1,), (1,)), ((), ())),
                         preferred_element_type=jnp.float32) + b1_ref[0]
    g = lax.dot_general(xb, w2_ref[0], (((1,), (1,)), ((), ())),
                        preferred_element_type=jnp.float32) + b2_ref[0]
    h = (h1 * jax.nn.sigmoid(h1) * g).astype(jnp.bfloat16)
    y = lax.dot_general(h, w3_ref[0], (((1,), (1,)), ((), ())),
                        preferred_element_type=jnp.float32) + b3_ref[0]
    ys_ref[...] = y


def _ffn(bmap, xs, w1b, w2b, w3b, b1, b2, b3):
    grid_spec = pltpu.PrefetchScalarGridSpec(
        num_scalar_prefetch=1,
        grid=(NBLK,),
        in_specs=[
            pl.BlockSpec((BLK, DM), lambda b, m: (b, 0)),
            pl.BlockSpec((1, DH, DM), lambda b, m: (m[b], 0, 0)),
            pl.BlockSpec((1, DH, DM), lambda b, m: (m[b], 0, 0)),
            pl.BlockSpec((1, DM, DH), lambda b, m: (m[b], 0, 0)),
            pl.BlockSpec((1, 1, DH), lambda b, m: (m[b], 0, 0)),
            pl.BlockSpec((1, 1, DH), lambda b, m: (m[b], 0, 0)),
            pl.BlockSpec((1, 1, DM), lambda b, m: (m[b], 0, 0)),
        ],
        out_specs=pl.BlockSpec((BLK, DM), lambda b, m: (b, 0)),
    )
    return pl.pallas_call(
        _ffn_body,
        grid_spec=grid_spec,
        out_shape=jax.ShapeDtypeStruct((NPAD, DM), jnp.float32),
    )(bmap, xs, w1b, w2b, w3b, b1, b2, b3)


def _sc_mesh():
    return plsc.VectorSubcoreMesh(core_axis_name="c", subcore_axis_name="s")


def _dispatch(x2d, pos):
    @functools.partial(
        pl.kernel,
        out_type=jax.ShapeDtypeStruct((NPAD, DM), jnp.float32),
        mesh=_sc_mesh(),
        scratch_types=[
            pltpu.VMEM((TPW,), jnp.int32),
            pltpu.VMEM((TPW,), jnp.int32),
            pltpu.VMEM((TPW, DM), jnp.float32),
            pltpu.SemaphoreType.DMA,
            pltpu.SemaphoreType.DMA,
        ],
    )
    def k(x_hbm, pos_hbm, xs_hbm, idx0_v, idx1_v, rows_v, sem0, sem1):
        wid = lax.axis_index("s") * 2 + lax.axis_index("c")
        base = wid * TPW
        pltpu.sync_copy(pos_hbm.at[pl.ds(base, TPW)], idx0_v)
        pltpu.sync_copy(pos_hbm.at[pl.ds(NTOK + base, TPW)], idx1_v)
        pltpu.sync_copy(x_hbm.at[pl.ds(base, TPW)], rows_v)
        c0 = pltpu.async_copy(rows_v, xs_hbm.at[idx0_v], sem0)
        c1 = pltpu.async_copy(rows_v, xs_hbm.at[idx1_v], sem1)
        c0.wait()
        c1.wait()

    return k(x2d, pos)


def _combine(ys, pos, wexp):
    @functools.partial(
        pl.kernel,
        out_type=jax.ShapeDtypeStruct((NTOK, DM), jnp.float32),
        mesh=_sc_mesh(),
        scratch_types=[
            pltpu.VMEM((TPW,), jnp.int32),
            pltpu.VMEM((TPW,), jnp.int32),
            pltpu.VMEM((TPW, 16), jnp.float32),
            pltpu.VMEM((TPW, 16), jnp.float32),
            pltpu.VMEM((TPW, DM), jnp.float32),
            pltpu.VMEM((TPW, DM), jnp.float32),
            pltpu.SemaphoreType.DMA,
            pltpu.SemaphoreType.DMA,
        ],
    )
    def k(ys_hbm, pos_hbm, wexp_hbm, out_hbm,
          idx0_v, idx1_v, w0_v, w1_v, y0_v, y1_v, sem0, sem1):
        wid = lax.axis_index("s") * 2 + lax.axis_index("c")
        base = wid * TPW
        pltpu.sync_copy(pos_hbm.at[pl.ds(base, TPW)], idx0_v)
        pltpu.sync_copy(pos_hbm.at[pl.ds(NTOK + base, TPW)], idx1_v)
        pltpu.sync_copy(wexp_hbm.at[pl.ds(base, TPW)], w0_v)
        pltpu.sync_copy(wexp_hbm.at[pl.ds(NTOK + base, TPW)], w1_v)
        g0 = pltpu.async_copy(ys_hbm.at[idx0_v], y0_v, sem0)
        g1 = pltpu.async_copy(ys_hbm.at[idx1_v], y1_v, sem1)
        g0.wait()
        g1.wait()

        def row(r, carry):
            w0 = w0_v[r, :]
            w1 = w1_v[r, :]
            for c in range(DM // 16):
                sl = pl.ds(c * 16, 16)
                y0_v[r, sl] = y0_v[r, sl] * w0 + y1_v[r, sl] * w1
            return carry

        lax.fori_loop(0, TPW, row, 0)
        pltpu.sync_copy(y0_v, out_hbm.at[pl.ds(base, TPW)])

    return k(ys, pos, wexp)


def kernel(x, Wr, W1, b1, W2, b2, W3, b3):
    b, t, d = x.shape
    x2d = x.reshape(b * t, d)
    pos2d, wexp, bmap2d, aux2d = _router(x2d, Wr)
    pos = pos2d.reshape(NA)
    bmap = bmap2d.reshape(NBLK)
    xs = _dispatch(x2d, pos)
    ys = _ffn(bmap, xs,
              W1.astype(jnp.bfloat16), W2.astype(jnp.bfloat16),
              W3.astype(jnp.bfloat16),
              b1.reshape(NE, 1, DH), b2.reshape(NE, 1, DH),
              b3.reshape(NE, 1, DM))
    out = _combine(ys, pos, wexp)
    return out.reshape(b, t, d), aux2d.reshape(())


# speedup vs baseline: 1.5923x; 1.2410x over previous
import functools

import jax
import jax.numpy as jnp
from jax import lax
from jax.experimental import pallas as pl
from jax.experimental.pallas import tpu as pltpu
from jax.experimental.pallas import tpu_sc as plsc

DM = 768
DH = 3072
NE = 8
NTOK = 2048
NA = 2 * NTOK
BLK = 128
NBLK = (NA + NE * (BLK - 1) + BLK - 1) // BLK
NPAD = NBLK * BLK
CH = 512
NW = 32
TPW = NTOK // NW


def _router_body(x_ref, wr_ref, pos_ref, wexp_ref, blk_ref, aux_ref):
    x = x_ref[...]
    wr = wr_ref[...]
    logits = lax.dot_general(x, wr, (((1,), (1,)), ((), ())),
                             preferred_element_type=jnp.float32)
    m = jnp.max(logits, axis=1, keepdims=True)
    ex = jnp.exp(logits - m)
    p = ex / jnp.sum(ex, axis=1, keepdims=True)

    idx = lax.broadcasted_iota(jnp.int32, (NTOK, NE), 1)
    v1 = jnp.max(p, axis=1, keepdims=True)
    i1 = jnp.min(jnp.where(p == v1, idx, NE), axis=1, keepdims=True)
    p2 = jnp.where(idx == i1, -1.0, p)
    v2 = jnp.max(p2, axis=1, keepdims=True)
    i2 = jnp.min(jnp.where(p2 == v2, idx, NE), axis=1, keepdims=True)
    s = v1 + v2 + 1e-9
    w1 = v1 / s
    w2 = v2 / s

    e1 = (idx == i1).astype(jnp.float32)
    e2 = (idx == i2).astype(jnp.float32)
    oh = jnp.concatenate([e1, e2], axis=0)

    ri = lax.broadcasted_iota(jnp.int32, (CH, CH), 0)
    ci = lax.broadcasted_iota(jnp.int32, (CH, CH), 1)
    tril = (ci < ri).astype(jnp.float32)
    tot = jnp.zeros((1, NE), jnp.float32)
    ranks = []
    for i in range(NA // CH):
        chunk = lax.slice(oh, (i * CH, 0), ((i + 1) * CH, NE))
        rk = lax.dot_general(tril, chunk, (((1,), (0,)), ((), ())),
                             preferred_element_type=jnp.float32) + tot
        ranks.append(rk)
        tot = tot + jnp.sum(chunk, axis=0, keepdims=True)
    rank = jnp.concatenate(ranks, axis=0)

    cnt = tot.astype(jnp.int32)
    pc = ((cnt + BLK - 1) // BLK) * BLK
    pcf = pc.astype(jnp.float32)
    er = lax.broadcasted_iota(jnp.int32, (NE, NE), 0)
    ec = lax.broadcasted_iota(jnp.int32, (NE, NE), 1)
    excl = (er < ec).astype(jnp.float32)
    offs = lax.dot_general(pcf, excl, (((1,), (0,)), ((), ())),
                           preferred_element_type=jnp.float32)

    posf = jnp.sum(oh * (rank + offs), axis=1, keepdims=True)
    pos_ref[...] = posf.astype(jnp.int32)

    w12 = jnp.concatenate([w1, w2], axis=0)
    wexp_ref[...] = jnp.broadcast_to(w12, (NA, 16))

    bno = lax.broadcasted_iota(jnp.int32, (NBLK, 1), 0)
    endb = (offs.astype(jnp.int32) + pc) // BLK
    emap = jnp.sum((bno >= endb).astype(jnp.int32), axis=1, keepdims=True)
    blk_ref[...] = jnp.minimum(emap, NE - 1)

    cnt1 = jnp.sum(e1, axis=0, keepdims=True)
    meanp = jnp.sum(p, axis=0, keepdims=True) * (1.0 / NTOK)
    aux_ref[...] = jnp.sum(cnt1 * meanp, axis=1, keepdims=True) * (
        float(NE) / NTOK)


def _router(x2d, wr):
    return pl.pallas_call(
        _router_body,
        out_shape=[
            jax.ShapeDtypeStruct((NA, 1), jnp.int32),
            jax.ShapeDtypeStruct((NA, 16), jnp.float32),
            jax.ShapeDtypeStruct((NBLK, 1), jnp.int32),
            jax.ShapeDtypeStruct((1, 1), jnp.float32),
        ],
    )(x2d, wr)


def _ffn_body(bmap_ref, xs_ref, w1_ref, w2_ref, w3_ref,
              b1_ref, b2_ref, b3_ref, ys_ref):
    del bmap_ref
    xb = xs_ref[...]
    h1 = lax.dot_general(xb, w1_ref[0], (((1,), (1,)), ((), ())),
                         preferred_element_type=jnp.float32) + b1_ref[0]
    g = lax.dot_general(xb, w2_ref[0], (((1,), (1,)), ((), ())),
                        preferred_element_type=jnp.float32) + b2_ref[0]
    h = h1 * jax.nn.sigmoid(h1) * g
    y = lax.dot_general(h, w3_ref[0], (((1,), (1,)), ((), ())),
                        preferred_element_type=jnp.float32) + b3_ref[0]
    ys_ref[...] = y


def _ffn(bmap, xs, w1b, w2b, w3b, b1, b2, b3):
    grid_spec = pltpu.PrefetchScalarGridSpec(
        num_scalar_prefetch=1,
        grid=(NBLK,),
        in_specs=[
            pl.BlockSpec((BLK, DM), lambda b, m: (b, 0)),
            pl.BlockSpec((1, DH, DM), lambda b, m: (m[b], 0, 0)),
            pl.BlockSpec((1, DH, DM), lambda b, m: (m[b], 0, 0)),
            pl.BlockSpec((1, DM, DH), lambda b, m: (m[b], 0, 0)),
            pl.BlockSpec((1, 1, DH), lambda b, m: (m[b], 0, 0)),
            pl.BlockSpec((1, 1, DH), lambda b, m: (m[b], 0, 0)),
            pl.BlockSpec((1, 1, DM), lambda b, m: (m[b], 0, 0)),
        ],
        out_specs=pl.BlockSpec((BLK, DM), lambda b, m: (b, 0)),
    )
    return pl.pallas_call(
        _ffn_body,
        grid_spec=grid_spec,
        out_shape=jax.ShapeDtypeStruct((NPAD, DM), jnp.float32),
        compiler_params=pltpu.CompilerParams(
            vmem_limit_bytes=100 * 1024 * 1024),
    )(bmap, xs, w1b, w2b, w3b, b1, b2, b3)


def _sc_mesh():
    return plsc.VectorSubcoreMesh(core_axis_name="c", subcore_axis_name="s")


def _dispatch(x2d, pos):
    @functools.partial(
        pl.kernel,
        out_type=jax.ShapeDtypeStruct((NPAD, DM), jnp.float32),
        mesh=_sc_mesh(),
        scratch_types=[
            pltpu.VMEM((TPW,), jnp.int32),
            pltpu.VMEM((TPW,), jnp.int32),
            pltpu.VMEM((TPW, DM), jnp.float32),
            pltpu.SemaphoreType.DMA,
            pltpu.SemaphoreType.DMA,
        ],
    )
    def k(x_hbm, pos_hbm, xs_hbm, idx0_v, idx1_v, rows_v, sem0, sem1):
        wid = lax.axis_index("s") * 2 + lax.axis_index("c")
        base = wid * TPW
        pltpu.sync_copy(pos_hbm.at[pl.ds(base, TPW)], idx0_v)
        pltpu.sync_copy(pos_hbm.at[pl.ds(NTOK + base, TPW)], idx1_v)
        pltpu.sync_copy(x_hbm.at[pl.ds(base, TPW)], rows_v)
        c0 = pltpu.async_copy(rows_v, xs_hbm.at[idx0_v], sem0)
        c1 = pltpu.async_copy(rows_v, xs_hbm.at[idx1_v], sem1)
        c0.wait()
        c1.wait()

    return k(x2d, pos)


def _combine(ys, pos, wexp):
    @functools.partial(
        pl.kernel,
        out_type=jax.ShapeDtypeStruct((NTOK, DM), jnp.float32),
        mesh=_sc_mesh(),
        scratch_types=[
            pltpu.VMEM((TPW,), jnp.int32),
            pltpu.VMEM((TPW,), jnp.int32),
            pltpu.VMEM((TPW, 16), jnp.float32),
            pltpu.VMEM((TPW, 16), jnp.float32),
            pltpu.VMEM((TPW, DM), jnp.float32),
            pltpu.VMEM((TPW, DM), jnp.float32),
            pltpu.SemaphoreType.DMA,
            pltpu.SemaphoreType.DMA,
        ],
    )
    def k(ys_hbm, pos_hbm, wexp_hbm, out_hbm,
          idx0_v, idx1_v, w0_v, w1_v, y0_v, y1_v, sem0, sem1):
        wid = lax.axis_index("s") * 2 + lax.axis_index("c")
        base = wid * TPW
        pltpu.sync_copy(pos_hbm.at[pl.ds(base, TPW)], idx0_v)
        pltpu.sync_copy(pos_hbm.at[pl.ds(NTOK + base, TPW)], idx1_v)
        pltpu.sync_copy(wexp_hbm.at[pl.ds(base, TPW)], w0_v)
        pltpu.sync_copy(wexp_hbm.at[pl.ds(NTOK + base, TPW)], w1_v)
        g0 = pltpu.async_copy(ys_hbm.at[idx0_v], y0_v, sem0)
        g1 = pltpu.async_copy(ys_hbm.at[idx1_v], y1_v, sem1)
        g0.wait()
        g1.wait()

        def row(r, carry):
            w0 = w0_v[r, :]
            w1 = w1_v[r, :]
            for c in range(DM // 16):
                sl = pl.ds(c * 16, 16)
                y0_v[r, sl] = y0_v[r, sl] * w0 + y1_v[r, sl] * w1
            return carry

        lax.fori_loop(0, TPW, row, 0)
        pltpu.sync_copy(y0_v, out_hbm.at[pl.ds(base, TPW)])

    return k(ys, pos, wexp)


def kernel(x, Wr, W1, b1, W2, b2, W3, b3):
    b, t, d = x.shape
    x2d = x.reshape(b * t, d)
    pos2d, wexp, bmap2d, aux2d = _router(x2d, Wr)
    pos = pos2d.reshape(NA)
    bmap = bmap2d.reshape(NBLK)
    xs = _dispatch(x2d, pos)
    ys = _ffn(bmap, xs, W1, W2, W3,
              b1.reshape(NE, 1, DH), b2.reshape(NE, 1, DH),
              b3.reshape(NE, 1, DM))
    out = _combine(ys, pos, wexp)
    return out.reshape(b, t, d), aux2d.reshape(())


# speedup vs baseline: 2.4946x; 1.5667x over previous
import functools

import jax
import jax.numpy as jnp
from jax import lax
from jax.experimental import pallas as pl
from jax.experimental.pallas import tpu as pltpu
from jax.experimental.pallas import tpu_sc as plsc

DM = 768
DH = 3072
NE = 8
NTOK = 2048
NA = 2 * NTOK
BLK = 256
NBLK = (NA + NE * (BLK - 1) + BLK - 1) // BLK
NPAD = NBLK * BLK
CH = 512
NW = 32
TPW = NTOK // NW


def _router_body(x_ref, wr_ref, pos_ref, wexp_ref, blk_ref, aux_ref):
    x = x_ref[...]
    wr = wr_ref[...]
    logits = lax.dot_general(x, wr, (((1,), (1,)), ((), ())),
                             preferred_element_type=jnp.float32)
    m = jnp.max(logits, axis=1, keepdims=True)
    ex = jnp.exp(logits - m)
    p = ex / jnp.sum(ex, axis=1, keepdims=True)

    idx = lax.broadcasted_iota(jnp.int32, (NTOK, NE), 1)
    v1 = jnp.max(p, axis=1, keepdims=True)
    i1 = jnp.min(jnp.where(p == v1, idx, NE), axis=1, keepdims=True)
    p2 = jnp.where(idx == i1, -1.0, p)
    v2 = jnp.max(p2, axis=1, keepdims=True)
    i2 = jnp.min(jnp.where(p2 == v2, idx, NE), axis=1, keepdims=True)
    s = v1 + v2 + 1e-9
    w1 = v1 / s
    w2 = v2 / s

    e1 = (idx == i1).astype(jnp.float32)
    e2 = (idx == i2).astype(jnp.float32)
    oh = jnp.concatenate([e1, e2], axis=0)

    ri = lax.broadcasted_iota(jnp.int32, (CH, CH), 0)
    ci = lax.broadcasted_iota(jnp.int32, (CH, CH), 1)
    tril = (ci < ri).astype(jnp.float32)
    tot = jnp.zeros((1, NE), jnp.float32)
    ranks = []
    for i in range(NA // CH):
        chunk = lax.slice(oh, (i * CH, 0), ((i + 1) * CH, NE))
        rk = lax.dot_general(tril, chunk, (((1,), (0,)), ((), ())),
                             preferred_element_type=jnp.float32) + tot
        ranks.append(rk)
        tot = tot + jnp.sum(chunk, axis=0, keepdims=True)
    rank = jnp.concatenate(ranks, axis=0)

    cnt = tot.astype(jnp.int32)
    pc = ((cnt + BLK - 1) // BLK) * BLK
    pcf = pc.astype(jnp.float32)
    er = lax.broadcasted_iota(jnp.int32, (NE, NE), 0)
    ec = lax.broadcasted_iota(jnp.int32, (NE, NE), 1)
    excl = (er < ec).astype(jnp.float32)
    offs = lax.dot_general(pcf, excl, (((1,), (0,)), ((), ())),
                           preferred_element_type=jnp.float32)

    posf = jnp.sum(oh * (rank + offs), axis=1, keepdims=True)
    pos_ref[...] = posf.astype(jnp.int32)

    w12 = jnp.concatenate([w1, w2], axis=0)
    wexp_ref[...] = jnp.broadcast_to(w12, (NA, 16))

    bno = lax.broadcasted_iota(jnp.int32, (NBLK, 1), 0)
    endb = (offs.astype(jnp.int32) + pc) // BLK
    emap = jnp.sum((bno >= endb).astype(jnp.int32), axis=1, keepdims=True)
    blk_ref[...] = jnp.minimum(emap, NE - 1)

    cnt1 = jnp.sum(e1, axis=0, keepdims=True)
    meanp = jnp.sum(p, axis=0, keepdims=True) * (1.0 / NTOK)
    aux_ref[...] = jnp.sum(cnt1 * meanp, axis=1, keepdims=True) * (
        float(NE) / NTOK)


def _router(x2d, wr):
    return pl.pallas_call(
        _router_body,
        out_shape=[
            jax.ShapeDtypeStruct((NA, 1), jnp.int32),
            jax.ShapeDtypeStruct((NA, 16), jnp.float32),
            jax.ShapeDtypeStruct((NBLK, 1), jnp.int32),
            jax.ShapeDtypeStruct((1, 1), jnp.float32),
        ],
    )(x2d, wr)


def _ffn_body(bm, xs_ref, w1h, w2h, w3h, b1_ref, b2_ref, b3_ref, ys_ref,
              w1s, w2s, w3s, w1c, w2c, w3c, sems):
    b = pl.program_id(0)
    mb = bm[b]
    prevm = bm[jnp.maximum(b - 1, 0)]
    chg = jnp.logical_or(b == 0, mb != prevm)

    qh = DH // 4
    qm = DM // 4

    def issue(e):
        for i, (wh, wv, q) in enumerate(
                ((w1h, w1s, qh), (w2h, w2s, qh), (w3h, w3s, qm))):
            for c in range(4):
                pltpu.make_async_copy(wh.at[e, pl.ds(c * q, q)],
                                      wv.at[pl.ds(c * q, q)],
                                      sems.at[i]).start()

    def wait_one(i, wh, wv, q):
        for c in range(4):
            pltpu.make_async_copy(wh.at[0, pl.ds(c * q, q)],
                                  wv.at[pl.ds(c * q, q)],
                                  sems.at[i]).wait()

    @pl.when(b == 0)
    def _():
        issue(mb)

    @pl.when(chg)
    def _():
        wait_one(0, w1h, w1s, qh)
        w1c[...] = w1s[...].astype(jnp.bfloat16)
        wait_one(1, w2h, w2s, qh)
        w2c[...] = w2s[...].astype(jnp.bfloat16)
        wait_one(2, w3h, w3s, qm)
        w3c[...] = w3s[...].astype(jnp.bfloat16)
        nxt = lax.while_loop(
            lambda j: jnp.logical_and(j < NBLK, bm[j] == mb),
            lambda j: j + 1, b + 1)

        @pl.when(nxt < NBLK)
        def _():
            issue(bm[nxt])

    xb = xs_ref[...].astype(jnp.bfloat16)
    h1 = lax.dot_general(xb, w1c[...], (((1,), (1,)), ((), ())),
                         preferred_element_type=jnp.float32) + b1_ref[0]
    g = lax.dot_general(xb, w2c[...], (((1,), (1,)), ((), ())),
                        preferred_element_type=jnp.float32) + b2_ref[0]
    h = (h1 * jax.nn.sigmoid(h1) * g).astype(jnp.bfloat16)
    y = lax.dot_general(h, w3c[...], (((1,), (1,)), ((), ())),
                        preferred_element_type=jnp.float32) + b3_ref[0]
    ys_ref[...] = y


def _ffn(bmap, xs, w1b, w2b, w3b, b1, b2, b3):
    grid_spec = pltpu.PrefetchScalarGridSpec(
        num_scalar_prefetch=1,
        grid=(NBLK,),
        in_specs=[
            pl.BlockSpec((BLK, DM), lambda b, m: (b, 0)),
            pl.BlockSpec(memory_space=pltpu.MemorySpace.HBM),
            pl.BlockSpec(memory_space=pltpu.MemorySpace.HBM),
            pl.BlockSpec(memory_space=pltpu.MemorySpace.HBM),
            pl.BlockSpec((1, 1, DH), lambda b, m: (m[b], 0, 0)),
            pl.BlockSpec((1, 1, DH), lambda b, m: (m[b], 0, 0)),
            pl.BlockSpec((1, 1, DM), lambda b, m: (m[b], 0, 0)),
        ],
        out_specs=pl.BlockSpec((BLK, DM), lambda b, m: (b, 0)),
        scratch_shapes=[
            pltpu.VMEM((DH, DM), jnp.float32),
            pltpu.VMEM((DH, DM), jnp.float32),
            pltpu.VMEM((DM, DH), jnp.float32),
            pltpu.VMEM((DH, DM), jnp.bfloat16),
            pltpu.VMEM((DH, DM), jnp.bfloat16),
            pltpu.VMEM((DM, DH), jnp.bfloat16),
            pltpu.SemaphoreType.DMA((3,)),
        ],
    )
    return pl.pallas_call(
        _ffn_body,
        grid_spec=grid_spec,
        out_shape=jax.ShapeDtypeStruct((NPAD, DM), jnp.float32),
        compiler_params=pltpu.CompilerParams(
            vmem_limit_bytes=100 * 1024 * 1024),
    )(bmap, xs, w1b, w2b, w3b, b1, b2, b3)


def _sc_mesh():
    return plsc.VectorSubcoreMesh(core_axis_name="c", subcore_axis_name="s")


def _dispatch(x2d, pos):
    @functools.partial(
        pl.kernel,
        out_type=jax.ShapeDtypeStruct((NPAD, DM), jnp.float32),
        mesh=_sc_mesh(),
        scratch_types=[
            pltpu.VMEM((TPW,), jnp.int32),
            pltpu.VMEM((TPW,), jnp.int32),
            pltpu.VMEM((TPW, DM), jnp.float32),
            pltpu.SemaphoreType.DMA,
            pltpu.SemaphoreType.DMA,
        ],
    )
    def k(x_hbm, pos_hbm, xs_hbm, idx0_v, idx1_v, rows_v, sem0, sem1):
        wid = lax.axis_index("s") * 2 + lax.axis_index("c")
        base = wid * TPW
        pltpu.sync_copy(pos_hbm.at[pl.ds(base, TPW)], idx0_v)
        pltpu.sync_copy(pos_hbm.at[pl.ds(NTOK + base, TPW)], idx1_v)
        pltpu.sync_copy(x_hbm.at[pl.ds(base, TPW)], rows_v)
        c0 = pltpu.async_copy(rows_v, xs_hbm.at[idx0_v], sem0)
        c1 = pltpu.async_copy(rows_v, xs_hbm.at[idx1_v], sem1)
        c0.wait()
        c1.wait()

    return k(x2d, pos)


def _combine(ys, pos, wexp):
    @functools.partial(
        pl.kernel,
        out_type=jax.ShapeDtypeStruct((NTOK, DM), jnp.float32),
        mesh=_sc_mesh(),
        scratch_types=[
            pltpu.VMEM((TPW,), jnp.int32),
            pltpu.VMEM((TPW,), jnp.int32),
            pltpu.VMEM((TPW, 16), jnp.float32),
            pltpu.VMEM((TPW, 16), jnp.float32),
            pltpu.VMEM((TPW, DM), jnp.float32),
            pltpu.VMEM((TPW, DM), jnp.float32),
            pltpu.SemaphoreType.DMA,
            pltpu.SemaphoreType.DMA,
        ],
    )
    def k(ys_hbm, pos_hbm, wexp_hbm, out_hbm,
          idx0_v, idx1_v, w0_v, w1_v, y0_v, y1_v, sem0, sem1):
        wid = lax.axis_index("s") * 2 + lax.axis_index("c")
        base = wid * TPW
        pltpu.sync_copy(pos_hbm.at[pl.ds(base, TPW)], idx0_v)
        pltpu.sync_copy(pos_hbm.at[pl.ds(NTOK + base, TPW)], idx1_v)
        pltpu.sync_copy(wexp_hbm.at[pl.ds(base, TPW)], w0_v)
        pltpu.sync_copy(wexp_hbm.at[pl.ds(NTOK + base, TPW)], w1_v)
        g0 = pltpu.async_copy(ys_hbm.at[idx0_v], y0_v, sem0)
        g1 = pltpu.async_copy(ys_hbm.at[idx1_v], y1_v, sem1)
        g0.wait()
        g1.wait()

        def row(r, carry):
            w0 = w0_v[r, :]
            w1 = w1_v[r, :]
            for c in range(DM // 16):
                sl = pl.ds(c * 16, 16)
                y0_v[r, sl] = y0_v[r, sl] * w0 + y1_v[r, sl] * w1
            return carry

        lax.fori_loop(0, TPW, row, 0)
        pltpu.sync_copy(y0_v, out_hbm.at[pl.ds(base, TPW)])

    return k(ys, pos, wexp)


def kernel(x, Wr, W1, b1, W2, b2, W3, b3):
    b, t, d = x.shape
    x2d = x.reshape(b * t, d)
    pos2d, wexp, bmap2d, aux2d = _router(x2d, Wr)
    pos = pos2d.reshape(NA)
    bmap = bmap2d.reshape(NBLK)
    xs = _dispatch(x2d, pos)
    ys = _ffn(bmap, xs, W1, W2, W3,
              b1.reshape(NE, 1, DH), b2.reshape(NE, 1, DH),
              b3.reshape(NE, 1, DM))
    out = _combine(ys, pos, wexp)
    return out.reshape(b, t, d), aux2d.reshape(())
